# baseline (speedup 1.0000x reference)
import functools

import jax
import jax.numpy as jnp
from jax import lax
from jax.experimental import pallas as pl
from jax.experimental.pallas import tpu as pltpu
from jax.experimental.pallas import tpu_sc as plsc

_BATCH = 4096
_HIST = 50
_D = 64
_NC = 2
_NS = 16
_NW = _NC * _NS
_RPW = _BATCH // _NW
_NB = 10
_NG = _HIST // _NB

_mesh = plsc.VectorSubcoreMesh(core_axis_name="c", subcore_axis_name="s")


@functools.partial(
    pl.kernel,
    mesh=_mesh,
    out_type=jax.ShapeDtypeStruct((_HIST, 8, _NW, 8, 128), jnp.float32),
    compiler_params=pltpu.CompilerParams(
        use_tc_tiling_on_sc=False, needs_layout_passes=False
    ),
    scratch_types=[
        pltpu.VMEM((_HIST, _RPW), jnp.int32),
        pltpu.VMEM((_NB * _RPW, _D), jnp.float32),
        pltpu.VMEM((2, _D, 128), jnp.float32),
    ] + [pltpu.SemaphoreType.DMA] * (_NB + 2),
)
def _gather(idx_hbm, table_hbm, out_hbm, idx_v, rows_v, tr_v, *sems):
    gsems = sems[:_NB]
    wsems = sems[_NB:]
    wid = lax.axis_index("s") * _NC + lax.axis_index("c")
    pltpu.sync_copy(idx_hbm.at[:, pl.ds(wid * _RPW, _RPW)], idx_v)

    def buf(b):
        return rows_v.at[pl.ds(b * _RPW, _RPW)]

    lanes = lax.iota(jnp.int32, 16)

    def transpose_chunk(b, tb):
        def d_body(d, carry):
            col = jnp.full((16,), 0, jnp.int32) + d
            for i in range(8):
                vals = plsc.load_gather(buf(b), [lanes + (16 * i), col])
                tr_v[tb, d, pl.ds(16 * i, 16)] = vals
            return carry

        lax.fori_loop(0, _D, d_body, 0)

    def emit_writes(h, tb):
        for kd in range(8):
            pltpu.async_copy(
                tr_v.at[tb, pl.ds(kd * 8, 8)], out_hbm.at[h, kd, wid], wsems[tb]
            )

    def drain_writes(tb):
        for kd in range(8):
            pltpu.make_async_copy(
                tr_v.at[tb, pl.ds(kd * 8, 8)], out_hbm.at[0, kd, 0], wsems[tb]
            ).wait()

    for b in range(_NB):
        pltpu.async_copy(table_hbm.at[idx_v.at[b]], buf(b), gsems[b])

    def grp(g, carry):
        for b in range(_NB):
            chunk = g * _NB + b
            tb = b % 2
            pltpu.make_async_copy(table_hbm.at[idx_v.at[b]], buf(b), gsems[b]).wait()

            @pl.when(chunk >= 2)
            def _():
                drain_writes(tb)

            transpose_chunk(b, tb)
            emit_writes(chunk, tb)

            @pl.when(chunk + _NB < _HIST)
            def _():
                pltpu.async_copy(
                    table_hbm.at[idx_v.at[chunk + _NB]], buf(b), gsems[b]
                )
        return carry

    lax.fori_loop(0, _NG, grp, 0)

    drain_writes(0)
    drain_writes(1)


def kernel(word_batch, table):
    wbt = word_batch.astype(jnp.int32).T
    out5 = _gather(wbt, table)
    return out5.transpose(2, 4, 0, 1, 3).reshape(_BATCH, _HIST, _D)
